# baseline (speedup 1.0000x reference)
import functools

import jax
import jax.numpy as jnp
from jax import lax
from jax.experimental import pallas as pl
from jax.experimental.pallas import tpu as pltpu
from jax.experimental.pallas import tpu_sc as plsc

DIM = 32
NC = 2
NS = 16
NW = NC * NS
CHUNK = 128
SB = 4


def _make_gather(B: int, T: int):
    n_blocks = B // CHUNK
    n_super = n_blocks // SB
    n_chunks = T * n_super
    chunks_per_w = n_chunks // NW
    rows_per_c = SB * CHUNK
    assert n_chunks % NW == 0 and chunks_per_w % 2 == 0
    mesh = plsc.VectorSubcoreMesh(core_axis_name="c", subcore_axis_name="s")

    @functools.partial(
        pl.kernel,
        out_type=jax.ShapeDtypeStruct((T, DIM // 8, n_blocks, 8, CHUNK),
                                      jnp.float32),
        mesh=mesh,
        scratch_types=[
            pltpu.VMEM((chunks_per_w, rows_per_c), jnp.int32),
            pltpu.VMEM((2, rows_per_c, DIM), jnp.float32),
            pltpu.VMEM((2, DIM // 8, SB, 8, CHUNK), jnp.float32),
            pltpu.SemaphoreType.DMA,
            pltpu.SemaphoreType.DMA,
            pltpu.SemaphoreType.DMA,
            pltpu.SemaphoreType.DMA,
        ],
        compiler_params=pltpu.CompilerParams(use_tc_tiling_on_sc=False,
                                             needs_layout_passes=False),
    )
    def gather_kernel(idx_hbm, table_hbm, out_hbm, idx_v, gbuf, tbuf,
                      gs0, gs1, ws0, ws1):
        gsem = (gs0, gs1)
        wsem = (ws0, ws1)
        wid = lax.axis_index("s") * NC + lax.axis_index("c")
        pltpu.sync_copy(idx_hbm.at[wid], idx_v)

        lane = lax.iota(jnp.int32, 16)
        rows = [lane + (bs * CHUNK + 16 * k)
                for bs in range(SB) for k in range(CHUNK // 16)]

        def fire_chunk(j, slot):
            for q in range(SB):
                pltpu.async_copy(
                    table_hbm.at[idx_v.at[j, pl.ds(q * CHUNK, CHUNK)]],
                    gbuf.at[slot, pl.ds(q * CHUNK, CHUNK)],
                    gsem[slot],
                )

        def drain_gathers(j, slot):
            for q in range(SB):
                pltpu.make_async_copy(
                    table_hbm.at[idx_v.at[j, pl.ds(q * CHUNK, CHUNK)]],
                    gbuf.at[slot, pl.ds(q * CHUNK, CHUNK)],
                    gsem[slot],
                ).wait()

        def drain_scatters(slot):
            for g in range(DIM // 8):
                pltpu.make_async_copy(
                    tbuf.at[slot, g],
                    out_hbm.at[0, g, pl.ds(0, SB)],
                    wsem[slot],
                ).wait()

        def transpose_chunk(slot):
            gref = gbuf.at[slot]

            def dbody(d4, carry):
                for dd in range(4):
                    d = d4 * 4 + dd
                    g = d // 8
                    d8 = d - g * 8
                    col = jnp.full((16,), d, jnp.int32)
                    for bs in range(SB):
                        for k in range(CHUNK // 16):
                            row = rows[bs * (CHUNK // 16) + k]
                            v = plsc.load_gather(gref, [row, col])
                            tbuf[slot, g, bs, d8, pl.ds(16 * k, 16)] = v
                return carry

            lax.fori_loop(0, DIM // 4, dbody, 0)

        def process_chunk(j, slot, drain):
            drain_gathers(j, slot)
            if drain:
                drain_scatters(slot)
            transpose_chunk(slot)
            cid = wid * chunks_per_w + j
            t = cid // n_super
            blk = (cid - t * n_super) * SB
            for g in range(DIM // 8):
                pltpu.async_copy(
                    tbuf.at[slot, g],
                    out_hbm.at[t, g, pl.ds(blk, SB)],
                    wsem[slot],
                )

        fire_chunk(0, 0)
        fire_chunk(1, 1)
        process_chunk(0, 0, drain=False)
        fire_chunk(2, 0)
        process_chunk(1, 1, drain=False)
        fire_chunk(3, 1)

        def body(i, carry):
            j = 2 + 2 * i
            process_chunk(j, 0, drain=True)
            fire_chunk(j + 2, 0)
            process_chunk(j + 1, 1, drain=True)
            fire_chunk(j + 3, 1)
            return carry

        lax.fori_loop(0, chunks_per_w // 2 - 2, body, 0)

        process_chunk(chunks_per_w - 2, 0, drain=True)
        process_chunk(chunks_per_w - 1, 1, drain=True)
        drain_scatters(0)
        drain_scatters(1)

    return gather_kernel


def kernel(x, E):
    b, t = x.shape
    idx = jnp.transpose(x).reshape(NW, (b * t) // (NW * SB * CHUNK),
                                   SB * CHUNK)
    out5 = _make_gather(b, t)(idx.astype(jnp.int32), E)
    return out5.transpose(2, 4, 0, 1, 3).reshape(b, t, DIM)


# speedup vs baseline: 1.5852x; 1.5852x over previous
import functools

import jax
import jax.numpy as jnp
from jax import lax
from jax.experimental import pallas as pl
from jax.experimental.pallas import tpu as pltpu
from jax.experimental.pallas import tpu_sc as plsc

DIM = 32
NC = 2
NS = 16
NW = NC * NS
CHUNK = 128
SB = 4
L = 16


def _make_gather(B: int, T: int):
    n_blocks = B // CHUNK
    n_super = n_blocks // SB
    n_chunks = T * n_super
    chunks_per_w = n_chunks // NW
    rows_per_c = SB * CHUNK
    tile_words = 8 * CHUNK
    sub_words = SB * tile_words
    assert n_chunks % NW == 0 and chunks_per_w % 2 == 0
    mesh = plsc.VectorSubcoreMesh(core_axis_name="c", subcore_axis_name="s")

    @functools.partial(
        pl.kernel,
        out_type=jax.ShapeDtypeStruct((T, DIM // 8, n_blocks * tile_words),
                                      jnp.float32),
        mesh=mesh,
        scratch_types=[
            pltpu.VMEM((chunks_per_w, rows_per_c), jnp.int32),
            pltpu.VMEM((2, rows_per_c, DIM), jnp.float32),
            pltpu.VMEM((2, (DIM // 8) * sub_words), jnp.float32),
            pltpu.SemaphoreType.DMA,
            pltpu.SemaphoreType.DMA,
            pltpu.SemaphoreType.DMA,
            pltpu.SemaphoreType.DMA,
        ],
        compiler_params=pltpu.CompilerParams(use_tc_tiling_on_sc=False,
                                             needs_layout_passes=False),
    )
    def gather_kernel(idx_hbm, table_hbm, out_hbm, idx_v, gbuf, tbuf,
                      gs0, gs1, ws0, ws1):
        gsem = (gs0, gs1)
        wsem = (ws0, ws1)
        wid = lax.axis_index("s") * NC + lax.axis_index("c")
        pltpu.sync_copy(idx_hbm.at[wid], idx_v)

        lane = lax.iota(jnp.int32, L)

        def fire_chunk(j, slot):
            for q in range(SB):
                pltpu.async_copy(
                    table_hbm.at[idx_v.at[j, pl.ds(q * CHUNK, CHUNK)]],
                    gbuf.at[slot, pl.ds(q * CHUNK, CHUNK)],
                    gsem[slot],
                )

        def drain_gathers(j, slot):
            for q in range(SB):
                pltpu.make_async_copy(
                    table_hbm.at[idx_v.at[j, pl.ds(q * CHUNK, CHUNK)]],
                    gbuf.at[slot, pl.ds(q * CHUNK, CHUNK)],
                    gsem[slot],
                ).wait()

        def drain_scatters(slot):
            for g in range(DIM // 8):
                pltpu.make_async_copy(
                    tbuf.at[slot, pl.ds(g * sub_words, sub_words)],
                    out_hbm.at[0, g, pl.ds(0, sub_words)],
                    wsem[slot],
                ).wait()

        def transpose_chunk(slot):
            gref = gbuf.at[slot]
            tref = tbuf.at[slot]

            def jbody(j, carry):
                cj = (lane + j) & (L - 1)
                for d0 in (0, L):
                    col = cj + d0
                    pos_a = (((col >> 3) * sub_words)
                             + ((col & 7) << 7) + lane)
                    for m in range(rows_per_c // L):
                        row = lane + m * L
                        pos = pos_a + ((m >> 3) * tile_words
                                       + (m & 7) * L)
                        v = plsc.load_gather(gref, [row, col])
                        plsc.store_scatter(tref, [pos], v)
                return carry

            lax.fori_loop(0, L, jbody, 0)

        def process_chunk(j, slot, drain):
            drain_gathers(j, slot)
            if drain:
                drain_scatters(slot)
            transpose_chunk(slot)
            cid = wid * chunks_per_w + j
            t = cid // n_super
            blk = (cid - t * n_super) * SB
            for g in range(DIM // 8):
                pltpu.async_copy(
                    tbuf.at[slot, pl.ds(g * sub_words, sub_words)],
                    out_hbm.at[t, g, pl.ds(blk * tile_words, sub_words)],
                    wsem[slot],
                )

        fire_chunk(0, 0)
        fire_chunk(1, 1)
        process_chunk(0, 0, drain=False)
        fire_chunk(2, 0)
        process_chunk(1, 1, drain=False)
        fire_chunk(3, 1)

        def body(i, carry):
            j = 2 + 2 * i
            process_chunk(j, 0, drain=True)
            fire_chunk(j + 2, 0)
            process_chunk(j + 1, 1, drain=True)
            fire_chunk(j + 3, 1)
            return carry

        lax.fori_loop(0, chunks_per_w // 2 - 2, body, 0)

        process_chunk(chunks_per_w - 2, 0, drain=True)
        process_chunk(chunks_per_w - 1, 1, drain=True)
        drain_scatters(0)
        drain_scatters(1)

    return gather_kernel


def kernel(x, E):
    b, t = x.shape
    idx = jnp.transpose(x).reshape(NW, (b * t) // (NW * SB * CHUNK),
                                   SB * CHUNK)
    out3 = _make_gather(b, t)(idx.astype(jnp.int32), E)
    out5 = out3.reshape(t, DIM // 8, b // CHUNK, 8, CHUNK)
    return out5.transpose(2, 4, 0, 1, 3).reshape(b, t, DIM)
